# baseline (speedup 1.0000x reference)
import functools

import jax
import jax.numpy as jnp
from jax import lax
from jax.experimental import pallas as pl
from jax.experimental.pallas import tpu as pltpu
from jax.experimental.pallas import tpu_sc as plsc

_LANES = 16
_NWORKERS = 32
_CHUNK = 4096


def kernel(categorical_inputs, tables):
    B, T = categorical_inputs.shape
    _, V, D = tables.shape

    tab_t = jnp.transpose(tables, (0, 2, 1))
    cat_t = categorical_inputs.T

    n_pairs = T * D // _NWORKERS

    mesh = plsc.VectorSubcoreMesh(core_axis_name="core", subcore_axis_name="subcore")

    @functools.partial(
        pl.kernel,
        out_type=jax.ShapeDtypeStruct((T, D, B), tables.dtype),
        mesh=mesh,
        compiler_params=pltpu.CompilerParams(needs_layout_passes=False),
        scratch_types=[
            pltpu.VMEM((V,), jnp.float32),
            pltpu.VMEM((B,), jnp.int32),
            pltpu.VMEM((2, _CHUNK), jnp.float32),
            pltpu.SemaphoreType.DMA,
        ],
    )
    def run(tab_hbm, cat_hbm, out_hbm, vrow, idxv, outb, osem):
        wid = lax.axis_index("subcore") * 2 + lax.axis_index("core")
        p0 = wid * n_pairs

        @pl.loop(0, n_pairs)
        def pair(i):
            p = p0 + i
            t = lax.shift_right_logical(p, 5)
            d = jnp.bitwise_and(p, 31)

            @pl.when(jnp.logical_or(d == 0, i == 0))
            def _():
                pltpu.sync_copy(cat_hbm.at[t, :], idxv)

            pltpu.sync_copy(tab_hbm.at[t, d, :], vrow)

            writes = []
            for c in range(B // _CHUNK):
                s = c % 2
                if c >= 2:
                    writes[c - 2].wait()

                @pl.loop(0, _CHUNK // _LANES, step=16)
                def gath(j):
                    ivs = [
                        idxv[pl.ds(c * _CHUNK + (j + u) * _LANES, _LANES)]
                        for u in range(16)
                    ]
                    gs = [plsc.load_gather(vrow, [iv]) for iv in ivs]
                    for u in range(16):
                        outb[s, pl.ds((j + u) * _LANES, _LANES)] = gs[u]

                writes.append(
                    pltpu.async_copy(
                        outb.at[s], out_hbm.at[t, d, pl.ds(c * _CHUNK, _CHUNK)], osem
                    )
                )
            writes[-2].wait()
            writes[-1].wait()

    out_t = run(tab_t, cat_t)
    return jnp.transpose(out_t, (2, 0, 1))


# speedup vs baseline: 1.0114x; 1.0114x over previous
import functools

import jax
import jax.numpy as jnp
from jax import lax
from jax.experimental import pallas as pl
from jax.experimental.pallas import tpu as pltpu
from jax.experimental.pallas import tpu_sc as plsc

_LANES = 16
_NWORKERS = 32
_CHUNK = 4096


def kernel(categorical_inputs, tables):
    B, T = categorical_inputs.shape
    _, V, D = tables.shape

    tab_t = jnp.transpose(tables, (0, 2, 1))
    cat_t = categorical_inputs.T

    n_pairs = T * D // _NWORKERS
    n_chunks = B // _CHUNK
    chunk_bytes = _CHUNK * 4

    mesh = plsc.VectorSubcoreMesh(core_axis_name="core", subcore_axis_name="subcore")

    @functools.partial(
        pl.kernel,
        out_type=jax.ShapeDtypeStruct((T, D, B), tables.dtype),
        mesh=mesh,
        compiler_params=pltpu.CompilerParams(needs_layout_passes=False),
        scratch_types=[
            pltpu.VMEM((V,), jnp.float32),
            pltpu.VMEM((B,), jnp.int32),
            pltpu.VMEM((2, _CHUNK), jnp.float32),
            pltpu.SemaphoreType.DMA,
            pltpu.SemaphoreType.DMA,
        ],
    )
    def run(tab_hbm, cat_hbm, out_hbm, vrow, idxv, outb, osem, rsem):
        wid = lax.axis_index("subcore") * 2 + lax.axis_index("core")
        p0 = wid * n_pairs

        @pl.loop(0, n_pairs)
        def pair(i):
            p = p0 + i
            t = lax.shift_right_logical(p, 5)
            d = jnp.bitwise_and(p, 31)

            rl = pltpu.make_async_copy(tab_hbm.at[t, d, :], vrow, rsem)
            rl.start()

            @pl.when(jnp.logical_or(d == 0, i == 0))
            def _():
                pltpu.sync_copy(cat_hbm.at[t, :], idxv)

            @pl.when(i > 0)
            def _():
                for s in range(2):
                    pltpu.make_async_copy(
                        tab_hbm.at[t, d, pl.ds(0, _CHUNK)], outb.at[s], osem
                    ).wait()

            rl.wait()

            writes = []
            for c in range(n_chunks):
                s = c % 2
                if c >= 2:
                    writes[c - 2].wait()

                @pl.loop(0, _CHUNK // _LANES, step=8)
                def gath(j):
                    ivs = [
                        idxv[pl.ds(c * _CHUNK + (j + u) * _LANES, _LANES)]
                        for u in range(8)
                    ]
                    gs = [plsc.load_gather(vrow, [iv]) for iv in ivs]
                    for u in range(8):
                        outb[s, pl.ds((j + u) * _LANES, _LANES)] = gs[u]

                writes.append(
                    pltpu.async_copy(
                        outb.at[s], out_hbm.at[t, d, pl.ds(c * _CHUNK, _CHUNK)], osem
                    )
                )

        for s in range(2):
            pltpu.make_async_copy(
                tab_hbm.at[0, 0, pl.ds(0, _CHUNK)], outb.at[s], osem
            ).wait()

    out_t = run(tab_t, cat_t)
    return jnp.transpose(out_t, (2, 0, 1))
